# baseline (speedup 1.0000x reference)
import functools

import jax
import jax.numpy as jnp
from jax import lax
from jax.experimental import pallas as pl
from jax.experimental.pallas import tpu as pltpu
from jax.experimental.pallas import tpu_sc as plsc

N = 10000
NP = 10240
E = 320000
EP = 327680
ROWS = EP // 128
RPT = ROWS // 16
NPT = NP // 16
BN = 1024
F = 128


def _pack_block(h):
    u = lax.bitcast_convert_type(h, jnp.int32)
    r = (u + jnp.int32(0x7FFF) + ((u >> 16) & 1)) >> 16
    parts = []
    for g in range(4):
        lo = r[:, 32 * g:32 * g + 16] & jnp.int32(0xFFFF)
        hi = r[:, 32 * g + 16:32 * g + 32] << 16
        parts.append(lo | hi)
    return jnp.concatenate(parts, axis=1)


def _tc1_body(x_ref, w_ref, asr_ref, adr_ref, h_ref, av_ref):
    xb = x_ref[...]
    h0 = jnp.dot(xb, w_ref[:, :F], preferred_element_type=jnp.float32)
    h1 = jnp.dot(xb, w_ref[:, F:], preferred_element_type=jnp.float32)
    h_ref[0, :, :] = _pack_block(h0)
    h_ref[1, :, :] = _pack_block(h1)
    hcat = jnp.concatenate([h0, h1], axis=1)
    dn = (((1,), (1,)), ((), ()))
    asr = lax.dot_general(asr_ref[...], hcat, dn,
                          preferred_element_type=jnp.float32)
    adr = lax.dot_general(adr_ref[...], hcat, dn,
                          preferred_element_type=jnp.float32)
    av_ref[...] = jnp.concatenate([asr, asr, adr, adr], axis=0)


def _tc1(xp, W1, a1_src, a1_dst):
    grid = NP // BN
    return pl.pallas_call(
        _tc1_body,
        grid=(grid,),
        in_specs=[
            pl.BlockSpec((BN, 128), lambda i: (i, 0)),
            pl.BlockSpec((128, 256), lambda i: (0, 0)),
            pl.BlockSpec((1, 256), lambda i: (0, 0)),
            pl.BlockSpec((1, 256), lambda i: (0, 0)),
        ],
        out_specs=[
            pl.BlockSpec((2, BN, 64), lambda i: (0, i, 0)),
            pl.BlockSpec((4, BN), lambda i: (0, i)),
        ],
        out_shape=[
            jax.ShapeDtypeStruct((2, NP, 64), jnp.int32),
            jax.ShapeDtypeStruct((4, NP), jnp.float32),
        ],
    )(xp, W1, a1_src.reshape(1, 256), a1_dst.reshape(1, 256))


def _tc2_body(h1_ref, wm_ref, wl_ref, ams_ref, amd_ref, als_ref, ald_ref,
              h2_ref, av_ref):
    p0 = h1_ref[0, :, :]
    p1 = h1_ref[1, :, :]
    hm = (jnp.dot(p0, wm_ref[:F, :], preferred_element_type=jnp.float32)
          + jnp.dot(p1, wm_ref[F:, :], preferred_element_type=jnp.float32))
    hl = (jnp.dot(p0, wl_ref[:F, :], preferred_element_type=jnp.float32)
          + jnp.dot(p1, wl_ref[F:, :], preferred_element_type=jnp.float32))
    h2_ref[0, :, :] = _pack_block(hm)
    h2_ref[1, :, :] = _pack_block(hl)
    dn = (((1,), (1,)), ((), ()))
    asm = lax.dot_general(ams_ref[...], hm, dn,
                          preferred_element_type=jnp.float32)
    adm = lax.dot_general(amd_ref[...], hm, dn,
                          preferred_element_type=jnp.float32)
    asl = lax.dot_general(als_ref[...], hl, dn,
                          preferred_element_type=jnp.float32)
    adl = lax.dot_general(ald_ref[...], hl, dn,
                          preferred_element_type=jnp.float32)
    av_ref[...] = jnp.concatenate([asm, asl, adm, adl], axis=0)


def _tc2(h1s, Wmu, Wls, amu_src, amu_dst, als_src, als_dst):
    grid = NP // BN
    return pl.pallas_call(
        _tc2_body,
        grid=(grid,),
        in_specs=[
            pl.BlockSpec((2, BN, F), lambda i: (0, i, 0)),
            pl.BlockSpec((256, F), lambda i: (0, 0)),
            pl.BlockSpec((256, F), lambda i: (0, 0)),
            pl.BlockSpec((1, F), lambda i: (0, 0)),
            pl.BlockSpec((1, F), lambda i: (0, 0)),
            pl.BlockSpec((1, F), lambda i: (0, 0)),
            pl.BlockSpec((1, F), lambda i: (0, 0)),
        ],
        out_specs=[
            pl.BlockSpec((2, BN, 64), lambda i: (0, i, 0)),
            pl.BlockSpec((4, BN), lambda i: (0, i)),
        ],
        out_shape=[
            jax.ShapeDtypeStruct((2, NP, 64), jnp.int32),
            jax.ShapeDtypeStruct((4, NP), jnp.float32),
        ],
    )(h1s, Wmu, Wls, amu_src.reshape(1, F), amu_dst.reshape(1, F),
      als_src.reshape(1, F), als_dst.reshape(1, F))


SCH = 16
NSC = RPT // SCH


def _sc_body(apply_relu, h_hbm, src_hbm, dst_hbm, av_hbm, m_hbm, b_hbm,
             out_hbm, src16, dst16, dga16, eav16, ebv16, wbuf,
             rowsA, rowsB, scalv, mv, bv, dnv, accs, dens,
             semEA, semEB, semFA, semFB, semRA, semRB):
    cid = lax.axis_index("c")
    sid = lax.axis_index("s")
    row0 = pl.multiple_of(sid * RPT, 8)
    nb0 = pl.multiple_of(sid * NPT, 128)
    coff = cid * NP
    aoff = (cid + 2) * NP

    @pl.loop(0, 128)
    def _zero(r):
        for g in range(8):
            scalv[r, pl.ds(16 * g, 16)] = jnp.zeros((16,), jnp.float32)

    @pl.loop(0, SCH)
    def _zerow(r):
        for g in range(8):
            wbuf[r, pl.ds(16 * g, 16)] = jnp.zeros((16,), jnp.float32)

    for j in range(NPT // 128):
        nb = pl.multiple_of(nb0 + j * 128, 128)
        pltpu.sync_copy(scalv, accs.at[pl.ds(nb, 128)])
        pltpu.sync_copy(wbuf.at[0], dens.at[pl.ds(nb, 128)])

    pltpu.sync_copy(m_hbm.at[pl.ds(pl.multiple_of(cid * 16, 16), 16)], mv)
    pltpu.sync_copy(b_hbm.at[pl.ds(pl.multiple_of(cid * 128, 128), 128)], bv)

    plsc.subcore_barrier()

    mvec = mv[...]

    def _issue(r, rows, semE, semF, semR):
        pltpu.async_copy(av_hbm.at[src16.at[r]], eav16.at[r], semE)
        pltpu.async_copy(av_hbm.at[dga16.at[r]], ebv16.at[r], semF)
        pltpu.async_copy(h_hbm.at[src16.at[r]], rows, semR)

    def _consume(r, rows, semE, semF, semR):
        pltpu.make_async_copy(av_hbm.at[src16.at[r]], eav16.at[r],
                              semE).wait()
        pltpu.make_async_copy(av_hbm.at[dga16.at[r]], ebv16.at[r],
                              semF).wait()
        pltpu.make_async_copy(h_hbm.at[src16.at[r]], rows, semR).wait()
        for g in range(8):
            sl = pl.ds(16 * g, 16)
            e = eav16[r, sl] + ebv16[r, sl]
            e = jnp.maximum(e, 0.2 * e)
            wbuf[r, sl] = jnp.exp(e - mvec)

        @plsc.parallel_loop(0, 8, unroll=1)
        def _scale(k):
            wg = wbuf[r, pl.ds(16 * k, 16)]
            for j in range(16):
                i = 16 * k + j
                wvec = jnp.full((16,), wg[j], jnp.float32)
                for g in range(4):
                    xi = rows[i, pl.ds(16 * g, 16)]
                    va = lax.bitcast_convert_type(xi << 16, jnp.float32)
                    vb = lax.bitcast_convert_type(xi & jnp.int32(-65536),
                                                  jnp.float32)
                    scalv[i, pl.ds(32 * g, 16)] = va * wvec
                    scalv[i, pl.ds(32 * g + 16, 16)] = vb * wvec

        pltpu.sync_copy(scalv, accs.at[dst16.at[r]], add=True)
        pltpu.sync_copy(wbuf.at[r], dens.at[dst16.at[r]], add=True)

    @pl.loop(0, NSC)
    def _super(s):
        r0 = pl.multiple_of(row0 + s * SCH, 8)
        pltpu.sync_copy(src_hbm.at[pl.ds(r0, SCH)], src16)
        pltpu.sync_copy(dst_hbm.at[pl.ds(r0, SCH)], dst16)

        @pl.loop(0, SCH)
        def _rebase(r):
            for g in range(8):
                sl = pl.ds(16 * g, 16)
                src16[r, sl] = src16[r, sl] + coff
                dga16[r, sl] = dst16[r, sl] + aoff

        _issue(0, rowsA, semEA, semFA, semRA)
        _issue(1, rowsB, semEB, semFB, semRB)

        @pl.loop(0, SCH // 2)
        def _pair(t):
            r = 2 * t
            _consume(r, rowsA, semEA, semFA, semRA)

            @pl.when(r + 2 < SCH)
            def _():
                _issue(r + 2, rowsA, semEA, semFA, semRA)

            _consume(r + 1, rowsB, semEB, semFB, semRB)

            @pl.when(r + 3 < SCH)
            def _():
                _issue(r + 3, rowsB, semEB, semFB, semRB)

    plsc.subcore_barrier()

    for j in range(NPT // 128):
        nb = pl.multiple_of(nb0 + j * 128, 128)
        pltpu.sync_copy(accs.at[pl.ds(nb, 128)], scalv)
        pltpu.sync_copy(dens.at[pl.ds(nb, 128)], dnv)

        @pl.loop(0, 8)
        def _norm(k):
            dg = dnv[pl.ds(16 * k, 16)] + 1e-16
            for j in range(16):
                i = 16 * k + j
                dvec = jnp.full((16,), dg[j], jnp.float32)
                for g in range(8):
                    sl = pl.ds(16 * g, 16)
                    v = scalv[i, sl] / dvec + bv[sl]
                    if apply_relu:
                        v = jnp.maximum(v, 0.0)
                    scalv[i, sl] = v

        pltpu.sync_copy(scalv, out_hbm.at[cid, pl.ds(nb, 128)])


def _sc_edge(apply_relu):
    mesh = plsc.VectorSubcoreMesh(core_axis_name="c", subcore_axis_name="s")
    return pl.kernel(
        functools.partial(_sc_body, apply_relu),
        out_type=jax.ShapeDtypeStruct((2, NP, F), jnp.float32),
        mesh=mesh,
        compiler_params=pltpu.CompilerParams(use_tc_tiling_on_sc=False),
        scratch_types=[
            pltpu.VMEM((SCH, 128), jnp.int32),
            pltpu.VMEM((SCH, 128), jnp.int32),
            pltpu.VMEM((SCH, 128), jnp.int32),
            pltpu.VMEM((SCH, 128), jnp.float32),
            pltpu.VMEM((SCH, 128), jnp.float32),
            pltpu.VMEM((SCH, 128), jnp.float32),
            pltpu.VMEM((128, 64), jnp.int32),
            pltpu.VMEM((128, 64), jnp.int32),
            pltpu.VMEM((128, 128), jnp.float32),
            pltpu.VMEM((16,), jnp.float32),
            pltpu.VMEM((128,), jnp.float32),
            pltpu.VMEM((128,), jnp.float32),
            pltpu.VMEM_SHARED((NP, F), jnp.float32),
            pltpu.VMEM_SHARED((NP,), jnp.float32),
        ] + [pltpu.SemaphoreType.DMA] * 6,
    )


_sc_edge_relu = _sc_edge(True)
_sc_edge_lin = _sc_edge(False)


def _shift(av):
    mm = jnp.max(av[0:2], axis=1) + jnp.max(av[2:4], axis=1)
    m = jnp.maximum(mm, 0.2 * mm)
    return jnp.broadcast_to(m[:, None], (2, 16))


def kernel(x, edge_index, W1, a1_src, a1_dst, b1, Wmu, amu_src, amu_dst, bmu,
           Wls, als_src, als_dst, bls):
    xp = jnp.pad(x, ((0, NP - N), (0, 0)))
    pad = jnp.full((EP - E,), NP - 1, jnp.int32)
    srcp = jnp.concatenate([edge_index[0], pad]).reshape(ROWS, 128)
    dstp = jnp.concatenate([edge_index[1], pad]).reshape(ROWS, 128)

    h1s, av1 = _tc1(xp, W1, a1_src, a1_dst)
    b1s = jnp.concatenate([b1[:F], b1[F:]])
    out1 = _sc_edge_relu(h1s.reshape(2 * NP, 64), srcp, dstp,
                         av1.reshape(4 * NP), _shift(av1).reshape(32), b1s)

    h2s, av2 = _tc2(out1, Wmu, Wls, amu_src, amu_dst, als_src, als_dst)
    b2s = jnp.concatenate([bmu, bls])
    out2 = _sc_edge_lin(h2s.reshape(2 * NP, 64), srcp, dstp,
                        av2.reshape(4 * NP), _shift(av2).reshape(32), b2s)

    return (out2[0, :N, :], out2[1, :N, :])


# speedup vs baseline: 1.0226x; 1.0226x over previous
import functools

import jax
import jax.numpy as jnp
from jax import lax
from jax.experimental import pallas as pl
from jax.experimental.pallas import tpu as pltpu
from jax.experimental.pallas import tpu_sc as plsc

N = 10000
NP = 10240
E = 320000
EP = 327680
ROWS = EP // 128
RPT = ROWS // 16
NPT = NP // 16
BN = 1024
F = 128


def _tc1_body(x_ref, w_ref, asr_ref, adr_ref, h_ref, av_ref):
    xb = x_ref[...]
    h0 = jnp.dot(xb, w_ref[:, :F], preferred_element_type=jnp.float32)
    h1 = jnp.dot(xb, w_ref[:, F:], preferred_element_type=jnp.float32)
    h_ref[0, :, :] = h0
    h_ref[1, :, :] = h1
    hcat = jnp.concatenate([h0, h1], axis=1)
    dn = (((1,), (1,)), ((), ()))
    asr = lax.dot_general(asr_ref[...], hcat, dn,
                          preferred_element_type=jnp.float32)
    adr = lax.dot_general(adr_ref[...], hcat, dn,
                          preferred_element_type=jnp.float32)
    av_ref[...] = jnp.concatenate([asr, asr, adr, adr], axis=0)


def _tc1(xp, W1, a1_src, a1_dst):
    grid = NP // BN
    return pl.pallas_call(
        _tc1_body,
        grid=(grid,),
        in_specs=[
            pl.BlockSpec((BN, 128), lambda i: (i, 0)),
            pl.BlockSpec((128, 256), lambda i: (0, 0)),
            pl.BlockSpec((1, 256), lambda i: (0, 0)),
            pl.BlockSpec((1, 256), lambda i: (0, 0)),
        ],
        out_specs=[
            pl.BlockSpec((2, BN, F), lambda i: (0, i, 0)),
            pl.BlockSpec((4, BN), lambda i: (0, i)),
        ],
        out_shape=[
            jax.ShapeDtypeStruct((2, NP, F), jnp.float32),
            jax.ShapeDtypeStruct((4, NP), jnp.float32),
        ],
    )(xp, W1, a1_src.reshape(1, 256), a1_dst.reshape(1, 256))


def _tc2_body(h1_ref, wm_ref, wl_ref, ams_ref, amd_ref, als_ref, ald_ref,
              h2_ref, av_ref):
    p0 = h1_ref[0, :, :]
    p1 = h1_ref[1, :, :]
    hm = (jnp.dot(p0, wm_ref[:F, :], preferred_element_type=jnp.float32)
          + jnp.dot(p1, wm_ref[F:, :], preferred_element_type=jnp.float32))
    hl = (jnp.dot(p0, wl_ref[:F, :], preferred_element_type=jnp.float32)
          + jnp.dot(p1, wl_ref[F:, :], preferred_element_type=jnp.float32))
    h2_ref[0, :, :] = hm
    h2_ref[1, :, :] = hl
    dn = (((1,), (1,)), ((), ()))
    asm = lax.dot_general(ams_ref[...], hm, dn,
                          preferred_element_type=jnp.float32)
    adm = lax.dot_general(amd_ref[...], hm, dn,
                          preferred_element_type=jnp.float32)
    asl = lax.dot_general(als_ref[...], hl, dn,
                          preferred_element_type=jnp.float32)
    adl = lax.dot_general(ald_ref[...], hl, dn,
                          preferred_element_type=jnp.float32)
    av_ref[...] = jnp.concatenate([asm, asl, adm, adl], axis=0)


def _tc2(h1s, Wmu, Wls, amu_src, amu_dst, als_src, als_dst):
    grid = NP // BN
    return pl.pallas_call(
        _tc2_body,
        grid=(grid,),
        in_specs=[
            pl.BlockSpec((2, BN, F), lambda i: (0, i, 0)),
            pl.BlockSpec((256, F), lambda i: (0, 0)),
            pl.BlockSpec((256, F), lambda i: (0, 0)),
            pl.BlockSpec((1, F), lambda i: (0, 0)),
            pl.BlockSpec((1, F), lambda i: (0, 0)),
            pl.BlockSpec((1, F), lambda i: (0, 0)),
            pl.BlockSpec((1, F), lambda i: (0, 0)),
        ],
        out_specs=[
            pl.BlockSpec((2, BN, F), lambda i: (0, i, 0)),
            pl.BlockSpec((4, BN), lambda i: (0, i)),
        ],
        out_shape=[
            jax.ShapeDtypeStruct((2, NP, F), jnp.float32),
            jax.ShapeDtypeStruct((4, NP), jnp.float32),
        ],
    )(h1s, Wmu, Wls, amu_src.reshape(1, F), amu_dst.reshape(1, F),
      als_src.reshape(1, F), als_dst.reshape(1, F))


SCH = 16
NSC = RPT // SCH


def _sc_body(apply_relu, h_hbm, src_hbm, dst_hbm, av_hbm, m_hbm, b_hbm,
             out_hbm, src16, dst16, dga16, eav16, ebv16, wbuf,
             rowsA, rowsB, scalv, mv, bv, dnv, accs, dens,
             semEA, semEB, semFA, semFB, semRA, semRB):
    cid = lax.axis_index("c")
    sid = lax.axis_index("s")
    row0 = pl.multiple_of(sid * RPT, 8)
    nb0 = pl.multiple_of(sid * NPT, 128)
    coff = cid * NP
    aoff = (cid + 2) * NP

    @pl.loop(0, 128)
    def _zero(r):
        for g in range(8):
            scalv[r, pl.ds(16 * g, 16)] = jnp.zeros((16,), jnp.float32)

    @pl.loop(0, SCH)
    def _zerow(r):
        for g in range(8):
            wbuf[r, pl.ds(16 * g, 16)] = jnp.zeros((16,), jnp.float32)

    for j in range(NPT // 128):
        nb = pl.multiple_of(nb0 + j * 128, 128)
        pltpu.sync_copy(scalv, accs.at[pl.ds(nb, 128)])
        pltpu.sync_copy(wbuf.at[0], dens.at[pl.ds(nb, 128)])

    pltpu.sync_copy(m_hbm.at[pl.ds(pl.multiple_of(cid * 16, 16), 16)], mv)
    pltpu.sync_copy(b_hbm.at[pl.ds(pl.multiple_of(cid * 128, 128), 128)], bv)

    plsc.subcore_barrier()

    mvec = mv[...]

    def _issue(r, rows, semE, semF, semR):
        pltpu.async_copy(av_hbm.at[src16.at[r]], eav16.at[r], semE)
        pltpu.async_copy(av_hbm.at[dga16.at[r]], ebv16.at[r], semF)
        pltpu.async_copy(h_hbm.at[src16.at[r]], rows, semR)

    def _consume(r, rows, semE, semF, semR):
        pltpu.make_async_copy(av_hbm.at[src16.at[r]], eav16.at[r],
                              semE).wait()
        pltpu.make_async_copy(av_hbm.at[dga16.at[r]], ebv16.at[r],
                              semF).wait()
        pltpu.make_async_copy(h_hbm.at[src16.at[r]], rows, semR).wait()
        for g in range(8):
            sl = pl.ds(16 * g, 16)
            e = eav16[r, sl] + ebv16[r, sl]
            e = jnp.maximum(e, 0.2 * e)
            wbuf[r, sl] = jnp.exp(e - mvec)

        @plsc.parallel_loop(0, 8, unroll=1)
        def _scale(k):
            wg = wbuf[r, pl.ds(16 * k, 16)]
            for j in range(16):
                i = 16 * k + j
                wvec = jnp.full((16,), wg[j], jnp.float32)
                for g in range(4):
                    xi = rows[i, pl.ds(16 * g, 16)]
                    va = lax.bitcast_convert_type(xi << 16, jnp.float32)
                    vb = lax.bitcast_convert_type(xi & jnp.int32(-65536),
                                                  jnp.float32)
                    scalv[i, pl.ds(32 * g, 16)] = va * wvec
                    scalv[i, pl.ds(32 * g + 16, 16)] = vb * wvec

        pltpu.sync_copy(scalv, accs.at[dst16.at[r]], add=True)
        pltpu.sync_copy(wbuf.at[r], dens.at[dst16.at[r]], add=True)

    @pl.loop(0, NSC)
    def _super(s):
        r0 = pl.multiple_of(row0 + s * SCH, 8)
        pltpu.sync_copy(src_hbm.at[pl.ds(r0, SCH)], src16)
        pltpu.sync_copy(dst_hbm.at[pl.ds(r0, SCH)], dst16)

        @pl.loop(0, SCH)
        def _rebase(r):
            for g in range(8):
                sl = pl.ds(16 * g, 16)
                src16[r, sl] = src16[r, sl] + coff
                dga16[r, sl] = dst16[r, sl] + aoff

        _issue(0, rowsA, semEA, semFA, semRA)
        _issue(1, rowsB, semEB, semFB, semRB)

        @pl.loop(0, SCH // 2)
        def _pair(t):
            r = 2 * t
            _consume(r, rowsA, semEA, semFA, semRA)

            @pl.when(r + 2 < SCH)
            def _():
                _issue(r + 2, rowsA, semEA, semFA, semRA)

            _consume(r + 1, rowsB, semEB, semFB, semRB)

            @pl.when(r + 3 < SCH)
            def _():
                _issue(r + 3, rowsB, semEB, semFB, semRB)

    plsc.subcore_barrier()

    for j in range(NPT // 128):
        nb = pl.multiple_of(nb0 + j * 128, 128)
        pltpu.sync_copy(accs.at[pl.ds(nb, 128)], scalv)
        pltpu.sync_copy(dens.at[pl.ds(nb, 128)], dnv)

        @pl.loop(0, 8)
        def _norm(k):
            dg = dnv[pl.ds(16 * k, 16)] + 1e-16
            for j in range(16):
                i = 16 * k + j
                dvec = jnp.full((16,), dg[j], jnp.float32)
                for g in range(8):
                    sl = pl.ds(16 * g, 16)
                    v = scalv[i, sl] / dvec + bv[sl]
                    if apply_relu:
                        v = jnp.maximum(v, 0.0)
                    scalv[i, sl] = v

        pltpu.sync_copy(scalv, out_hbm.at[cid, pl.ds(nb, 128)])


def _sc_edge(apply_relu):
    mesh = plsc.VectorSubcoreMesh(core_axis_name="c", subcore_axis_name="s")
    return pl.kernel(
        functools.partial(_sc_body, apply_relu),
        out_type=jax.ShapeDtypeStruct((2, NP, F), jnp.float32),
        mesh=mesh,
        compiler_params=pltpu.CompilerParams(use_tc_tiling_on_sc=False),
        scratch_types=[
            pltpu.VMEM((SCH, 128), jnp.int32),
            pltpu.VMEM((SCH, 128), jnp.int32),
            pltpu.VMEM((SCH, 128), jnp.int32),
            pltpu.VMEM((SCH, 128), jnp.float32),
            pltpu.VMEM((SCH, 128), jnp.float32),
            pltpu.VMEM((SCH, 128), jnp.float32),
            pltpu.VMEM((128, 64), jnp.int32),
            pltpu.VMEM((128, 64), jnp.int32),
            pltpu.VMEM((128, 128), jnp.float32),
            pltpu.VMEM((16,), jnp.float32),
            pltpu.VMEM((128,), jnp.float32),
            pltpu.VMEM((128,), jnp.float32),
            pltpu.VMEM_SHARED((NP, F), jnp.float32),
            pltpu.VMEM_SHARED((NP,), jnp.float32),
        ] + [pltpu.SemaphoreType.DMA] * 6,
    )


_sc_edge_relu = _sc_edge(True)
_sc_edge_lin = _sc_edge(False)


def _bfshuf(h):
    t = h.reshape(2 * NP, 4, 2, 16).swapaxes(2, 3)
    tb = t.astype(jnp.bfloat16).reshape(2 * NP, 64, 2)
    return lax.bitcast_convert_type(tb, jnp.int32)


def _shift(av):
    mm = jnp.max(av[0:2], axis=1) + jnp.max(av[2:4], axis=1)
    m = jnp.maximum(mm, 0.2 * mm)
    return jnp.broadcast_to(m[:, None], (2, 16))


def kernel(x, edge_index, W1, a1_src, a1_dst, b1, Wmu, amu_src, amu_dst, bmu,
           Wls, als_src, als_dst, bls):
    xp = jnp.pad(x, ((0, NP - N), (0, 0)))
    pad = jnp.full((EP - E,), NP - 1, jnp.int32)
    srcp = jnp.concatenate([edge_index[0], pad]).reshape(ROWS, 128)
    dstp = jnp.concatenate([edge_index[1], pad]).reshape(ROWS, 128)

    h1s, av1 = _tc1(xp, W1, a1_src, a1_dst)
    b1s = jnp.concatenate([b1[:F], b1[F:]])
    out1 = _sc_edge_relu(_bfshuf(h1s), srcp, dstp,
                         av1.reshape(4 * NP), _shift(av1).reshape(32), b1s)

    h2s, av2 = _tc2(out1, Wmu, Wls, amu_src, amu_dst, als_src, als_dst)
    b2s = jnp.concatenate([bmu, bls])
    out2 = _sc_edge_lin(_bfshuf(h2s), srcp, dstp,
                        av2.reshape(4 * NP), _shift(av2).reshape(32), b2s)

    return (out2[0, :N, :], out2[1, :N, :])
